# baseline (speedup 1.0000x reference)
import functools

import jax
import jax.numpy as jnp
from jax import lax
from jax.experimental import pallas as pl
from jax.experimental.pallas import tpu as pltpu
from jax.experimental.pallas import tpu_sc as plsc

N = 10000
E = 320000
D = 128

NC = 2
NS = 16
NW = NC * NS
L = 16

CHUNK = 64
KG = 8
NBLK = 640
E_PAD = NBLK * KG * CHUNK
NG0 = 24
NG1 = 16
N_PAD = 10240
DUMMY = N
ROWS_PER_TILE = N_PAD // NS

_mesh = plsc.VectorSubcoreMesh(core_axis_name="c", subcore_axis_name="s")


@functools.partial(
    pl.kernel,
    out_type=[
        jax.ShapeDtypeStruct((NC, N_PAD, D), jnp.float32),
        jax.ShapeDtypeStruct((NC, NS, N_PAD), jnp.float32),
    ],
    mesh=_mesh,
    compiler_params=pltpu.CompilerParams(needs_layout_passes=False),
    scratch_types=[
        pltpu.VMEM((KG, CHUNK), jnp.int32),
        pltpu.VMEM((KG, CHUNK), jnp.int32),
        pltpu.VMEM((CHUNK, D), jnp.float32),
        pltpu.VMEM((CHUNK, D), jnp.float32),
        pltpu.VMEM((N_PAD,), jnp.float32),
        pltpu.VMEM_SHARED((N_PAD, D), jnp.float32),
        pltpu.SemaphoreType.DMA,
        pltpu.SemaphoreType.DMA,
    ],
)
def _sc_segment_sum(x_hbm, srcs_hbm, dsts_hbm, zrow_hbm, znode_hbm,
                    acc_out, deg_out,
                    src_v, dst_v, row_a, row_b, hist_v, acc_sh, sem_a, sem_b):
    cid = lax.axis_index("c")
    sid = lax.axis_index("s")
    base = sid * ROWS_PER_TILE

    pltpu.sync_copy(zrow_hbm, row_a)
    pltpu.sync_copy(znode_hbm, hist_v)

    for t in range(10):
        pltpu.sync_copy(row_a, acc_sh.at[pl.ds(base + t * CHUNK, CHUNK)])

    plsc.subcore_barrier()

    ones16 = jnp.ones((L,), jnp.float32)
    bufs = (row_a, row_b)
    gsems = (sem_a, sem_b)

    def make_group(blk_base):
        def group(g, carry):
            blk = blk_base + g
            pltpu.sync_copy(srcs_hbm.at[blk], src_v)
            pltpu.sync_copy(dsts_hbm.at[blk], dst_v)
            gd = [None] * KG
            gd[0] = pltpu.async_copy(x_hbm.at[src_v.at[0]], bufs[0], gsems[0])
            for j in range(KG):
                gd[j].wait()
                if j + 1 < KG:
                    gd[j + 1] = pltpu.async_copy(
                        x_hbm.at[src_v.at[j + 1]], bufs[(j + 1) % 2],
                        gsems[(j + 1) % 2])
                pltpu.sync_copy(bufs[j % 2], acc_sh.at[dst_v.at[j]], add=True)
                for i in range(CHUNK // L):
                    idx16 = dst_v[j, pl.ds(i * L, L)]
                    plsc.addupdate_scatter(hist_v, [idx16], ones16)
            return carry
        return group

    @pl.when(cid == 0)
    def _():
        lax.fori_loop(0, NG0, make_group(sid * NG0), 0)

    @pl.when(cid == 1)
    def _():
        lax.fori_loop(0, NG1, make_group(NS * NG0 + sid * NG1), 0)

    plsc.subcore_barrier()

    for t in range(10):
        pltpu.sync_copy(acc_sh.at[pl.ds(base + t * CHUNK, CHUNK)], row_a)
        pltpu.sync_copy(row_a, acc_out.at[cid, pl.ds(base + t * CHUNK, CHUNK)])
    pltpu.sync_copy(hist_v, deg_out.at[cid, sid])


def _tc_body(x_ref, a0_ref, a1_ref, dT_ref, wsum_ref, w2t_ref, bc_ref, o_ref):
    s = a0_ref[...] + a1_ref[...]
    deg = jnp.sum(dT_ref[...], axis=1, keepdims=True)
    mean = s / jnp.maximum(deg, 1.0)
    o_ref[...] = (
        jnp.dot(x_ref[...], wsum_ref[...], precision=lax.Precision.HIGHEST,
                preferred_element_type=jnp.float32)
        - jnp.dot(mean, w2t_ref[...], precision=lax.Precision.HIGHEST,
                  preferred_element_type=jnp.float32)
        + bc_ref[...]
    )


def _tc_combine(x, a0, a1, deg_t, wsum_t, w2_t, bc):
    BR = 1000
    return pl.pallas_call(
        _tc_body,
        grid=(N // BR,),
        in_specs=[
            pl.BlockSpec((BR, D), lambda i: (i, 0)),
            pl.BlockSpec((BR, D), lambda i: (i, 0)),
            pl.BlockSpec((BR, D), lambda i: (i, 0)),
            pl.BlockSpec((BR, NW), lambda i: (i, 0)),
            pl.BlockSpec((D, D), lambda i: (0, 0)),
            pl.BlockSpec((D, D), lambda i: (0, 0)),
            pl.BlockSpec((1, D), lambda i: (0, 0)),
        ],
        out_specs=pl.BlockSpec((BR, D), lambda i: (i, 0)),
        out_shape=jax.ShapeDtypeStruct((N, D), jnp.float32),
    )(x, a0, a1, deg_t, wsum_t, w2_t, bc)


def kernel(x, edge_index, W1, b1, W2, b2):
    src = edge_index[0].astype(jnp.int32)
    dst = edge_index[1].astype(jnp.int32)
    pad = E_PAD - E
    src = jnp.concatenate([src, jnp.zeros((pad,), jnp.int32)])
    dst = jnp.concatenate([dst, jnp.full((pad,), DUMMY, jnp.int32)])
    srcs = src.reshape(NBLK, KG, CHUNK)
    dsts = dst.reshape(NBLK, KG, CHUNK)

    zrow = jnp.zeros((CHUNK, D), jnp.float32)
    znode = jnp.zeros((N_PAD,), jnp.float32)

    acc, deg = _sc_segment_sum(x, srcs, dsts, zrow, znode)

    deg_t = deg.reshape(NC * NS, N_PAD).T
    wsum_t = (W1 + W2).T
    w2_t = W2.T
    bc = (b1 + b2).reshape(1, D)
    return _tc_combine(x, acc[0], acc[1], deg_t, wsum_t, w2_t, bc)


# speedup vs baseline: 1.1684x; 1.1684x over previous
import functools

import jax
import jax.numpy as jnp
from jax import lax
from jax.experimental import pallas as pl
from jax.experimental.pallas import tpu as pltpu
from jax.experimental.pallas import tpu_sc as plsc

N = 10000
E = 320000
D = 128

NC = 2
NS = 16
NW = NC * NS
L = 16

CHUNK = 64
KG = 8
NBLK = 640
E_PAD = NBLK * KG * CHUNK
NG0 = 32
NG1 = 8
N_PAD = 10240
DUMMY = N
ROWS_PER_TILE = N_PAD // NS

_mesh = plsc.VectorSubcoreMesh(core_axis_name="c", subcore_axis_name="s")


@functools.partial(
    pl.kernel,
    out_type=[
        jax.ShapeDtypeStruct((NC, N_PAD, D), jnp.float32),
        jax.ShapeDtypeStruct((NC, NS, N_PAD), jnp.float32),
    ],
    mesh=_mesh,
    compiler_params=pltpu.CompilerParams(needs_layout_passes=False),
    scratch_types=[
        pltpu.VMEM((KG, CHUNK), jnp.int32),
        pltpu.VMEM((KG, CHUNK), jnp.int32),
        pltpu.VMEM((CHUNK, D), jnp.float32),
        pltpu.VMEM((CHUNK, D), jnp.float32),
        pltpu.VMEM((N_PAD,), jnp.float32),
        pltpu.VMEM_SHARED((N_PAD, D), jnp.float32),
        pltpu.SemaphoreType.DMA,
        pltpu.SemaphoreType.DMA,
    ],
)
def _sc_segment_sum(x_hbm, srcs_hbm, dsts_hbm, zrow_hbm, znode_hbm,
                    acc_out, deg_out,
                    src_v, dst_v, row_a, row_b, hist_v, acc_sh, sem_a, sem_b):
    cid = lax.axis_index("c")
    sid = lax.axis_index("s")
    base = sid * ROWS_PER_TILE

    pltpu.sync_copy(zrow_hbm, row_a)
    pltpu.sync_copy(znode_hbm, hist_v)

    for t in range(10):
        pltpu.sync_copy(row_a, acc_sh.at[pl.ds(base + t * CHUNK, CHUNK)])

    plsc.subcore_barrier()

    ones16 = jnp.ones((L,), jnp.float32)
    bufs = (row_a, row_b)
    gsems = (sem_a, sem_b)

    def make_group(blk_base):
        def group(g, carry):
            blk = blk_base + g
            pltpu.sync_copy(srcs_hbm.at[blk], src_v)
            pltpu.sync_copy(dsts_hbm.at[blk], dst_v)
            gd = [None] * KG
            gd[0] = pltpu.async_copy(x_hbm.at[src_v.at[0]], bufs[0], gsems[0])
            for j in range(KG):
                gd[j].wait()
                if j + 1 < KG:
                    gd[j + 1] = pltpu.async_copy(
                        x_hbm.at[src_v.at[j + 1]], bufs[(j + 1) % 2],
                        gsems[(j + 1) % 2])
                pltpu.sync_copy(bufs[j % 2], acc_sh.at[dst_v.at[j]], add=True)
                for i in range(CHUNK // L):
                    idx16 = dst_v[j, pl.ds(i * L, L)]
                    plsc.addupdate_scatter(hist_v, [idx16], ones16)
            return carry
        return group

    @pl.when(cid == 0)
    def _():
        lax.fori_loop(0, NG0, make_group(sid * NG0), 0)

    @pl.when(cid == 1)
    def _():
        lax.fori_loop(0, NG1, make_group(NS * NG0 + sid * NG1), 0)

    plsc.subcore_barrier()

    for t in range(10):
        pltpu.sync_copy(acc_sh.at[pl.ds(base + t * CHUNK, CHUNK)], row_a)
        pltpu.sync_copy(row_a, acc_out.at[cid, pl.ds(base + t * CHUNK, CHUNK)])
    pltpu.sync_copy(hist_v, deg_out.at[cid, sid])


def _tc_body(x_ref, a0_ref, a1_ref, dT_ref, wsum_ref, w2t_ref, bc_ref, o_ref):
    s = a0_ref[...] + a1_ref[...]
    deg = jnp.sum(dT_ref[...], axis=1, keepdims=True)
    mean = s / jnp.maximum(deg, 1.0)
    o_ref[...] = (
        jnp.dot(x_ref[...], wsum_ref[...], precision=lax.Precision.HIGHEST,
                preferred_element_type=jnp.float32)
        - jnp.dot(mean, w2t_ref[...], precision=lax.Precision.HIGHEST,
                  preferred_element_type=jnp.float32)
        + bc_ref[...]
    )


def _tc_combine(x, a0, a1, deg_t, wsum_t, w2_t, bc):
    BR = 1000
    return pl.pallas_call(
        _tc_body,
        grid=(N // BR,),
        in_specs=[
            pl.BlockSpec((BR, D), lambda i: (i, 0)),
            pl.BlockSpec((BR, D), lambda i: (i, 0)),
            pl.BlockSpec((BR, D), lambda i: (i, 0)),
            pl.BlockSpec((BR, NW), lambda i: (i, 0)),
            pl.BlockSpec((D, D), lambda i: (0, 0)),
            pl.BlockSpec((D, D), lambda i: (0, 0)),
            pl.BlockSpec((1, D), lambda i: (0, 0)),
        ],
        out_specs=pl.BlockSpec((BR, D), lambda i: (i, 0)),
        out_shape=jax.ShapeDtypeStruct((N, D), jnp.float32),
    )(x, a0, a1, deg_t, wsum_t, w2_t, bc)


def kernel(x, edge_index, W1, b1, W2, b2):
    src = edge_index[0].astype(jnp.int32)
    dst = edge_index[1].astype(jnp.int32)
    pad = E_PAD - E
    src = jnp.concatenate([src, jnp.zeros((pad,), jnp.int32)])
    dst = jnp.concatenate([dst, jnp.full((pad,), DUMMY, jnp.int32)])
    srcs = src.reshape(NBLK, KG, CHUNK)
    dsts = dst.reshape(NBLK, KG, CHUNK)

    zrow = jnp.zeros((CHUNK, D), jnp.float32)
    znode = jnp.zeros((N_PAD,), jnp.float32)

    acc, deg = _sc_segment_sum(x, srcs, dsts, zrow, znode)

    deg_t = deg.reshape(NC * NS, N_PAD).T
    wsum_t = (W1 + W2).T
    w2_t = W2.T
    bc = (b1 + b2).reshape(1, D)
    return _tc_combine(x, acc[0], acc[1], deg_t, wsum_t, w2_t, bc)
